# Initial kernel scaffold; baseline (speedup 1.0000x reference)
import jax
import jax.numpy as jnp
from jax.experimental import pallas as pl


def kernel(h, x, edges, params):
    raise NotImplementedError("write your pallas kernel here")


# speedup vs baseline: 3.0355x; 3.0355x over previous
import jax
import jax.numpy as jnp
from jax import lax
from jax.experimental import pallas as pl
from jax.experimental.pallas import tpu as pltpu
from jax.experimental.pallas import tpu_sc as plsc

NN = 10000
EE = 320000
H = 128
CW = 144
NC, NS = 2, 16
NW = NC * NS
EW = EE // NW
CH = 80
NSTEP = EW // CH
NPAD = 10240
RPT = NPAD // NS
NB = 2000
BE = 5000

f32 = jnp.float32


def _mesh():
    return plsc.VectorSubcoreMesh(core_axis_name="c", subcore_axis_name="s",
                                  num_cores=NC, num_subcores=NS)


_SC_PARAMS = pltpu.CompilerParams(use_tc_tiling_on_sc=False)


def _sc_gather_body(t1_hbm, t2_hbm, rowr_hbm, colr_hbm, out_hbm,
                    ridx, cidx, b10, b20, b11, b21,
                    gs0, gs1, ws0, ws1):
    cid = lax.axis_index("c")
    sid = lax.axis_index("s")
    wid = sid * NC + cid
    cb = wid * NSTEP
    pltpu.sync_copy(rowr_hbm.at[pl.ds(cb, NSTEP)], ridx)
    pltpu.sync_copy(colr_hbm.at[pl.ds(cb, NSTEP)], cidx)
    bufs = ((b10, b20), (b11, b21))
    gsems = (gs0, gs1)
    wsems = (ws0, ws1)

    def fire(k, s):
        b1, b2 = bufs[s]
        pltpu.async_copy(t1_hbm.at[ridx.at[k]], b1, gsems[s])
        pltpu.async_copy(t2_hbm.at[cidx.at[k]], b2, gsems[s])

    def waitg(s):
        b1, b2 = bufs[s]
        pltpu.make_async_copy(t1_hbm.at[ridx.at[0]], b1, gsems[s]).wait()
        pltpu.make_async_copy(t1_hbm.at[ridx.at[0]], b2, gsems[s]).wait()

    def compute(s):
        b1, b2 = bufs[s]

        def rowf(r, carry):
            for c in range(8):
                sl = pl.ds(c * 16, 16)
                b1[r, sl] = b1[r, sl] + b2[r, sl]
            sc = pl.ds(128, 16)
            b1[r, sc] = b1[r, sc] - b2[r, sc]
            return carry

        lax.fori_loop(0, CH, rowf, 0)

    def firew(k, s):
        b1, _ = bufs[s]
        pltpu.async_copy(b1, out_hbm.at[pl.ds((cb + k) * CH, CH)], wsems[s])

    def waitw(s):
        b1, _ = bufs[s]
        pltpu.make_async_copy(b1, out_hbm.at[pl.ds(cb * CH, CH)],
                              wsems[s]).wait()

    fire(0, 0)
    fire(1, 1)
    waitg(0)
    compute(0)
    firew(0, 0)

    def body(j, carry):
        k1 = 2 * j + 1
        waitw(0)
        fire(k1 + 1, 0)
        waitg(1)
        compute(1)
        firew(k1, 1)
        k2 = k1 + 1
        waitw(1)
        fire(k2 + 1, 1)
        waitg(0)
        compute(0)
        firew(k2, 0)
        return carry

    lax.fori_loop(0, (NSTEP - 3) // 2, body, 0)
    waitw(0)
    fire(NSTEP - 1, 0)
    waitg(1)
    compute(1)
    firew(NSTEP - 2, 1)
    waitg(0)
    compute(0)
    firew(NSTEP - 1, 0)
    waitw(1)
    waitw(0)


def _sc_gather(t1, t2, rowr, colr):
    f = pl.kernel(
        _sc_gather_body,
        out_type=jax.ShapeDtypeStruct((EE, CW), f32),
        mesh=_mesh(),
        scratch_types=(
            pltpu.VMEM((NSTEP, CH), jnp.int32),
            pltpu.VMEM((NSTEP, CH), jnp.int32),
            pltpu.VMEM((CH, CW), f32),
            pltpu.VMEM((CH, CW), f32),
            pltpu.VMEM((CH, CW), f32),
            pltpu.VMEM((CH, CW), f32),
            pltpu.SemaphoreType.DMA,
            pltpu.SemaphoreType.DMA,
            pltpu.SemaphoreType.DMA,
            pltpu.SemaphoreType.DMA,
        ),
        compiler_params=_SC_PARAMS,
    )
    return f(t1, t2, rowr, colr)


def _sc_scatter_body(eo_hbm, rowr_hbm, zc_hbm, acc_out,
                     ridx, mb0, mb1, ls0, ls1, acc_sh):
    cid = lax.axis_index("c")
    sid = lax.axis_index("s")
    wid = sid * NC + cid
    rb = sid * RPT
    pltpu.sync_copy(zc_hbm.at[pl.ds(rb, RPT)], acc_sh.at[pl.ds(rb, RPT)])
    cb = wid * NSTEP
    pltpu.sync_copy(rowr_hbm.at[pl.ds(cb, NSTEP)], ridx)
    plsc.subcore_barrier()
    bufs = (mb0, mb1)
    lsems = (ls0, ls1)

    def load(k, s):
        pltpu.async_copy(eo_hbm.at[pl.ds((cb + k) * CH, CH)], bufs[s],
                         lsems[s])

    def waitl(s):
        pltpu.make_async_copy(eo_hbm.at[pl.ds(cb * CH, CH)], bufs[s],
                              lsems[s]).wait()

    def scat(k, s):
        pltpu.sync_copy(bufs[s], acc_sh.at[ridx.at[k]], add=True)

    load(0, 0)
    load(1, 1)
    waitl(0)
    scat(0, 0)
    load(2, 0)

    def body(j, carry):
        k1 = 2 * j + 1
        waitl(1)
        scat(k1, 1)
        load(k1 + 2, 1)
        k2 = k1 + 1
        waitl(0)
        scat(k2, 0)
        load(k2 + 2, 0)
        return carry

    lax.fori_loop(0, (NSTEP - 3) // 2, body, 0)
    waitl(1)
    scat(NSTEP - 2, 1)
    waitl(0)
    scat(NSTEP - 1, 0)
    plsc.subcore_barrier()
    pltpu.sync_copy(acc_sh.at[pl.ds(rb, RPT)], acc_out.at[cid, pl.ds(rb, RPT)])


def _sc_scatter(eo, rowr, zc):
    f = pl.kernel(
        _sc_scatter_body,
        out_type=jax.ShapeDtypeStruct((NC, NPAD, CW), f32),
        mesh=_mesh(),
        scratch_types=(
            pltpu.VMEM((NSTEP, CH), jnp.int32),
            pltpu.VMEM((CH, CW), f32),
            pltpu.VMEM((CH, CW), f32),
            pltpu.SemaphoreType.DMA,
            pltpu.SemaphoreType.DMA,
            pltpu.VMEM_SHARED((NPAD, CW), f32),
        ),
        compiler_params=_SC_PARAMS,
    )
    return f(eo, rowr, zc)


def _edge_tc_body(gcat, w1r, b1, W2, b2, wa, ba, Wc1, bc1, wc2, eo_o):
    g = gcat[...][:, :H]
    cd = gcat[...][:, H:CW]
    radial = jnp.sum(cd * cd, axis=1, keepdims=True)
    m1 = g + radial * w1r[...] + b1[...]
    ms = m1 * jax.nn.sigmoid(m1)
    t = ms @ W2[...] + b2[...]
    m2 = t * jax.nn.sigmoid(t)
    att = jax.nn.sigmoid(jnp.sum(m2 * wa[...], axis=1, keepdims=True)
                         + ba[...][:, 0:1])
    mo = m2 * att
    u = mo @ Wc1[...] + bc1[...]
    cw = u * jax.nn.sigmoid(u)
    cws = jnp.sum(cw * wc2[...], axis=1, keepdims=True)
    one3 = (lax.broadcasted_iota(jnp.int32, (1, 16), 1) == 3).astype(f32)
    tr16 = cd * cws + one3
    eo_o[...] = jnp.concatenate([mo, tr16], axis=1)


def _edge_tc(gcat, wts):
    be = lambda i: (i, 0)
    w0 = lambda i: (0, 0)
    return pl.pallas_call(
        _edge_tc_body,
        grid=(EE // BE,),
        in_specs=[
            pl.BlockSpec((BE, CW), be),
            pl.BlockSpec((1, H), w0),
            pl.BlockSpec((1, H), w0),
            pl.BlockSpec((H, H), w0),
            pl.BlockSpec((1, H), w0),
            pl.BlockSpec((1, H), w0),
            pl.BlockSpec((1, H), w0),
            pl.BlockSpec((H, H), w0),
            pl.BlockSpec((1, H), w0),
            pl.BlockSpec((1, H), w0),
        ],
        out_specs=pl.BlockSpec((BE, CW), be),
        out_shape=jax.ShapeDtypeStruct((EE, CW), f32),
    )(gcat, *wts)


def _node_tc_body(hh, cp, a0, a1, Wn1a, Wn1b, bn1, Wn2, bn2, W1a, W1b,
                  hh_o, cp_o, t1_o, t2_o):
    acc = a0[0] + a1[0]
    agg = acc[:, :H]
    s16 = acc[:, H:CW]
    deg = jnp.clip(s16[:, 3:4], 1.0, None)
    mask3 = (lax.broadcasted_iota(jnp.int32, (1, 16), 1) < 3).astype(f32)
    cpn = cp[...] + (s16 / deg) * mask3
    t = hh[...] @ Wn1a[...] + agg @ Wn1b[...] + bn1[...]
    t = t * jax.nn.sigmoid(t)
    hn = hh[...] + t @ Wn2[...] + bn2[...]
    hh_o[...] = hn
    cp_o[...] = cpn
    t1_o[...] = jnp.concatenate([hn @ W1a[...], cpn], axis=1)
    t2_o[...] = jnp.concatenate([hn @ W1b[...], cpn], axis=1)


def _node_tc(hh, cp, acc, w, wn):
    nb = lambda i: (i, 0)
    a0 = lambda i: (0, i, 0)
    a1 = lambda i: (1, i, 0)
    w0 = lambda i: (0, 0)
    return pl.pallas_call(
        _node_tc_body,
        grid=(NN // NB,),
        in_specs=[
            pl.BlockSpec((NB, H), nb),
            pl.BlockSpec((NB, 16), nb),
            pl.BlockSpec((1, NB, CW), a0),
            pl.BlockSpec((1, NB, CW), a1),
            pl.BlockSpec((H, H), w0),
            pl.BlockSpec((H, H), w0),
            pl.BlockSpec((1, H), w0),
            pl.BlockSpec((H, H), w0),
            pl.BlockSpec((1, H), w0),
            pl.BlockSpec((H, H), w0),
            pl.BlockSpec((H, H), w0),
        ],
        out_specs=[pl.BlockSpec((NB, H), nb), pl.BlockSpec((NB, 16), nb),
                   pl.BlockSpec((NB, CW), nb), pl.BlockSpec((NB, CW), nb)],
        out_shape=[jax.ShapeDtypeStruct((NN, H), f32),
                   jax.ShapeDtypeStruct((NN, 16), f32),
                   jax.ShapeDtypeStruct((NN, CW), f32),
                   jax.ShapeDtypeStruct((NN, CW), f32)],
    )(hh, cp, acc, acc, w["Wn1a"], w["Wn1b"], w["bn1"],
      w["Wn2"], w["bn2"], wn["W1a"], wn["W1b"])


def _node_final_body(hh, a0, a1, Wn1a, Wn1b, bn1, Wn2, bn2, Weo, beo, ps_o):
    agg = a0[0][:, :H] + a1[0][:, :H]
    t = hh[...] @ Wn1a[...] + agg @ Wn1b[...] + bn1[...]
    t = t * jax.nn.sigmoid(t)
    hn = hh[...] + t @ Wn2[...] + bn2[...]
    eo = hn @ Weo[...] + beo[...]

    @pl.when(pl.program_id(0) == 0)
    def _():
        ps_o[...] = jnp.zeros((1, H), f32)

    ps_o[...] += jnp.sum(eo, axis=0, keepdims=True)


def _node_final(hh, acc, w, Weo, beo):
    nb = lambda i: (i, 0)
    a0 = lambda i: (0, i, 0)
    a1 = lambda i: (1, i, 0)
    w0 = lambda i: (0, 0)
    return pl.pallas_call(
        _node_final_body,
        grid=(NN // NB,),
        in_specs=[
            pl.BlockSpec((NB, H), nb),
            pl.BlockSpec((1, NB, CW), a0),
            pl.BlockSpec((1, NB, CW), a1),
            pl.BlockSpec((H, H), w0),
            pl.BlockSpec((H, H), w0),
            pl.BlockSpec((1, H), w0),
            pl.BlockSpec((H, H), w0),
            pl.BlockSpec((1, H), w0),
            pl.BlockSpec((H, H), w0),
            pl.BlockSpec((1, H), w0),
        ],
        out_specs=pl.BlockSpec((1, H), w0),
        out_shape=jax.ShapeDtypeStruct((1, H), f32),
    )(hh, acc, acc, w["Wn1a"], w["Wn1b"], w["bn1"], w["Wn2"], w["bn2"],
      Weo, beo)


def _emb_body(h_in, xp, We, be_, W1a, W1b, hh_o, t1_o, t2_o):
    hh = h_in[...] @ We[...] + be_[...]
    hh_o[...] = hh
    t1_o[...] = jnp.concatenate([hh @ W1a[...], xp[...]], axis=1)
    t2_o[...] = jnp.concatenate([hh @ W1b[...], xp[...]], axis=1)


def _emb(h, xp, We, be_, W1a, W1b):
    nb = lambda i: (i, 0)
    w0 = lambda i: (0, 0)
    return pl.pallas_call(
        _emb_body,
        grid=(NN // NB,),
        in_specs=[
            pl.BlockSpec((NB, H), nb),
            pl.BlockSpec((NB, 16), nb),
            pl.BlockSpec((H, H), w0),
            pl.BlockSpec((1, H), w0),
            pl.BlockSpec((H, H), w0),
            pl.BlockSpec((H, H), w0),
        ],
        out_specs=[pl.BlockSpec((NB, H), nb), pl.BlockSpec((NB, CW), nb),
                   pl.BlockSpec((NB, CW), nb)],
        out_shape=[jax.ShapeDtypeStruct((NN, H), f32),
                   jax.ShapeDtypeStruct((NN, CW), f32),
                   jax.ShapeDtypeStruct((NN, CW), f32)],
    )(h, xp, We, be_, W1a, W1b)


def _head_body(ps, Wm1, bm1, Wm2, bm2, o_ref):
    hm = ps[...] * (1.0 / NN)
    z = hm @ Wm1[...] + bm1[...]
    z = jnp.maximum(z, 0.0)
    o_ref[...] = z @ Wm2[...] + bm2[...]


def _head(ps, Wm1, bm1, Wm2, bm2):
    w0 = lambda i: (0, 0)
    return pl.pallas_call(
        _head_body,
        grid=(1,),
        in_specs=[
            pl.BlockSpec((1, H), w0),
            pl.BlockSpec((H, H), w0),
            pl.BlockSpec((1, H), w0),
            pl.BlockSpec((H, H), w0),
            pl.BlockSpec((1, H), w0),
        ],
        out_specs=pl.BlockSpec((1, H), w0),
        out_shape=jax.ShapeDtypeStruct((1, H), f32),
    )(ps, Wm1, bm1, Wm2, bm2)


def _layer_views(p):
    W1 = p["edge1"]["W"]
    return dict(
        W1a=W1[:H],
        W1b=W1[H:2 * H],
        ew=(W1[2 * H:2 * H + 1],
            p["edge1"]["b"].reshape(1, H),
            p["edge2"]["W"],
            p["edge2"]["b"].reshape(1, H),
            p["att"]["W"].T,
            jnp.broadcast_to(p["att"]["b"].reshape(1, 1), (1, H)),
            p["coord1"]["W"],
            p["coord1"]["b"].reshape(1, H),
            p["coord2"]["W"].T),
        Wn1a=p["node1"]["W"][:H],
        Wn1b=p["node1"]["W"][H:],
        bn1=p["node1"]["b"].reshape(1, H),
        Wn2=p["node2"]["W"],
        bn2=p["node2"]["b"].reshape(1, H),
    )


def kernel(h, x, edges, params):
    rowr = edges[0].reshape(NW * NSTEP, CH)
    colr = edges[1].reshape(NW * NSTEP, CH)
    xp = jnp.pad(x, ((0, 0), (0, 13)))
    zc = jnp.zeros((NPAD, CW), f32)
    lws = [_layer_views(p) for p in params["layers"]]
    hh, t1, t2 = _emb(h, xp, params["emb_in"]["W"],
                      params["emb_in"]["b"].reshape(1, H),
                      lws[0]["W1a"], lws[0]["W1b"])
    cp = xp
    ps = None
    for l in range(4):
        gcat = _sc_gather(t1, t2, rowr, colr)
        eo = _edge_tc(gcat, lws[l]["ew"])
        acc = _sc_scatter(eo, rowr, zc)
        if l < 3:
            hh, cp, t1, t2 = _node_tc(hh, cp, acc, lws[l], lws[l + 1])
        else:
            ps = _node_final(hh, acc, lws[l], params["emb_out"]["W"],
                             params["emb_out"]["b"].reshape(1, H))
    o = _head(ps, params["mlp1"]["W"], params["mlp1"]["b"].reshape(1, H),
              jnp.pad(params["mlp2"]["W"], ((0, 0), (0, H - 20))),
              jnp.pad(params["mlp2"]["b"], (0, H - 20)).reshape(1, H))
    return o[0, :20]


# speedup vs baseline: 3.3499x; 1.1036x over previous
import jax
import jax.numpy as jnp
from jax import lax
from jax.experimental import pallas as pl
from jax.experimental.pallas import tpu as pltpu
from jax.experimental.pallas import tpu_sc as plsc

NN = 10000
EE = 320000
H = 128
CW = 144
NC, NS = 2, 16
NW = NC * NS
EW = EE // NW
CH = 80
NSTEP = EW // CH
NPAD = 10240
RPT = NPAD // NS
NB = 2000
BE = 5000

f32 = jnp.float32


def _mesh():
    return plsc.VectorSubcoreMesh(core_axis_name="c", subcore_axis_name="s",
                                  num_cores=NC, num_subcores=NS)


_SC_PARAMS = pltpu.CompilerParams(use_tc_tiling_on_sc=False)


def _sc_gather_body(t1_hbm, t2_hbm, rowr_hbm, colr_hbm, m1_hbm, cd_hbm,
                    ridx, cidx, b10, b20, ob0, cdb0, b11, b21, ob1, cdb1,
                    gs0, gs1, ws0, ws1):
    cid = lax.axis_index("c")
    sid = lax.axis_index("s")
    wid = sid * NC + cid
    cb = wid * NSTEP
    pltpu.sync_copy(rowr_hbm.at[pl.ds(cb, NSTEP)], ridx)
    pltpu.sync_copy(colr_hbm.at[pl.ds(cb, NSTEP)], cidx)
    bufs = ((b10, b20, ob0, cdb0), (b11, b21, ob1, cdb1))
    gsems = (gs0, gs1)
    wsems = (ws0, ws1)

    def fire(k, s):
        b1, b2, _, _ = bufs[s]
        pltpu.async_copy(t1_hbm.at[ridx.at[k]], b1, gsems[s])
        pltpu.async_copy(t2_hbm.at[cidx.at[k]], b2, gsems[s])

    def waitg(s):
        b1, b2, _, _ = bufs[s]
        pltpu.make_async_copy(t1_hbm.at[ridx.at[0]], b1, gsems[s]).wait()
        pltpu.make_async_copy(t1_hbm.at[ridx.at[0]], b2, gsems[s]).wait()

    def compute(s):
        b1, b2, ob, cdb = bufs[s]

        def rowf(r, carry):
            for c in range(8):
                sl = pl.ds(c * 16, 16)
                ob[r, sl] = b1[r, sl] + b2[r, sl]
            sc = pl.ds(H, 16)
            cdb[r, pl.ds(0, 16)] = b1[r, sc] - b2[r, sc]
            return carry

        lax.fori_loop(0, CH, rowf, 0)

    def firew(k, s):
        _, _, ob, cdb = bufs[s]
        pltpu.async_copy(ob, m1_hbm.at[pl.ds((cb + k) * CH, CH)], wsems[s])
        pltpu.async_copy(cdb, cd_hbm.at[pl.ds((cb + k) * CH, CH)], wsems[s])

    def waitw(s):
        _, _, ob, cdb = bufs[s]
        pltpu.make_async_copy(ob, m1_hbm.at[pl.ds(cb * CH, CH)],
                              wsems[s]).wait()
        pltpu.make_async_copy(cdb, cd_hbm.at[pl.ds(cb * CH, CH)],
                              wsems[s]).wait()

    fire(0, 0)
    fire(1, 1)
    waitg(0)
    compute(0)
    firew(0, 0)

    def body(j, carry):
        k1 = 2 * j + 1
        waitw(0)
        fire(k1 + 1, 0)
        waitg(1)
        compute(1)
        firew(k1, 1)
        k2 = k1 + 1
        waitw(1)
        fire(k2 + 1, 1)
        waitg(0)
        compute(0)
        firew(k2, 0)
        return carry

    lax.fori_loop(0, (NSTEP - 3) // 2, body, 0)
    waitw(0)
    fire(NSTEP - 1, 0)
    waitg(1)
    compute(1)
    firew(NSTEP - 2, 1)
    waitg(0)
    compute(0)
    firew(NSTEP - 1, 0)
    waitw(1)
    waitw(0)


def _sc_gather(t1, t2, rowr, colr):
    f = pl.kernel(
        _sc_gather_body,
        out_type=(
            jax.ShapeDtypeStruct((EE, H), f32),
            jax.ShapeDtypeStruct((EE, 16), f32),
        ),
        mesh=_mesh(),
        scratch_types=(
            pltpu.VMEM((NSTEP, CH), jnp.int32),
            pltpu.VMEM((NSTEP, CH), jnp.int32),
            pltpu.VMEM((CH, CW), f32),
            pltpu.VMEM((CH, CW), f32),
            pltpu.VMEM((CH, H), f32),
            pltpu.VMEM((CH, 16), f32),
            pltpu.VMEM((CH, CW), f32),
            pltpu.VMEM((CH, CW), f32),
            pltpu.VMEM((CH, H), f32),
            pltpu.VMEM((CH, 16), f32),
            pltpu.SemaphoreType.DMA,
            pltpu.SemaphoreType.DMA,
            pltpu.SemaphoreType.DMA,
            pltpu.SemaphoreType.DMA,
        ),
        compiler_params=_SC_PARAMS,
    )
    return f(t1, t2, rowr, colr)


def _sc_scatter_body(mo_hbm, tr_hbm, rowr_hbm, z128, z16, acc_out, s_out,
                     ridx, mb0, tb0, mb1, tb1, ls0, ls1, acc_sh, s_sh):
    cid = lax.axis_index("c")
    sid = lax.axis_index("s")
    wid = sid * NC + cid
    rb = sid * RPT
    pltpu.sync_copy(z128.at[pl.ds(rb, RPT)], acc_sh.at[pl.ds(rb, RPT)])
    pltpu.sync_copy(z16.at[pl.ds(rb, RPT)], s_sh.at[pl.ds(rb, RPT)])
    cb = wid * NSTEP
    pltpu.sync_copy(rowr_hbm.at[pl.ds(cb, NSTEP)], ridx)
    plsc.subcore_barrier()
    bufs = ((mb0, tb0), (mb1, tb1))
    lsems = (ls0, ls1)

    def load(k, s):
        mb, tb = bufs[s]
        pltpu.async_copy(mo_hbm.at[pl.ds((cb + k) * CH, CH)], mb, lsems[s])
        pltpu.async_copy(tr_hbm.at[pl.ds((cb + k) * CH, CH)], tb, lsems[s])

    def waitl(s):
        mb, tb = bufs[s]
        pltpu.make_async_copy(mo_hbm.at[pl.ds(cb * CH, CH)], mb,
                              lsems[s]).wait()
        pltpu.make_async_copy(tr_hbm.at[pl.ds(cb * CH, CH)], tb,
                              lsems[s]).wait()

    def scat(k, s):
        mb, tb = bufs[s]
        pltpu.sync_copy(mb, acc_sh.at[ridx.at[k]], add=True)
        pltpu.sync_copy(tb, s_sh.at[ridx.at[k]], add=True)

    load(0, 0)
    load(1, 1)
    waitl(0)
    scat(0, 0)
    load(2, 0)

    def body(j, carry):
        k1 = 2 * j + 1
        waitl(1)
        scat(k1, 1)
        load(k1 + 2, 1)
        k2 = k1 + 1
        waitl(0)
        scat(k2, 0)
        load(k2 + 2, 0)
        return carry

    lax.fori_loop(0, (NSTEP - 3) // 2, body, 0)
    waitl(1)
    scat(NSTEP - 2, 1)
    waitl(0)
    scat(NSTEP - 1, 0)
    plsc.subcore_barrier()
    pltpu.sync_copy(acc_sh.at[pl.ds(rb, RPT)], acc_out.at[cid, pl.ds(rb, RPT)])
    pltpu.sync_copy(s_sh.at[pl.ds(rb, RPT)], s_out.at[cid, pl.ds(rb, RPT)])


def _sc_scatter(mo, tr, rowr, z128, z16):
    f = pl.kernel(
        _sc_scatter_body,
        out_type=(
            jax.ShapeDtypeStruct((NC, NPAD, H), f32),
            jax.ShapeDtypeStruct((NC, NPAD, 16), f32),
        ),
        mesh=_mesh(),
        scratch_types=(
            pltpu.VMEM((NSTEP, CH), jnp.int32),
            pltpu.VMEM((CH, H), f32),
            pltpu.VMEM((CH, 16), f32),
            pltpu.VMEM((CH, H), f32),
            pltpu.VMEM((CH, 16), f32),
            pltpu.SemaphoreType.DMA,
            pltpu.SemaphoreType.DMA,
            pltpu.VMEM_SHARED((NPAD, H), f32),
            pltpu.VMEM_SHARED((NPAD, 16), f32),
        ),
        compiler_params=_SC_PARAMS,
    )
    return f(mo, tr, rowr, z128, z16)


def _edge_tc_body(m1_in, cd_in, w1r, b1, W2, b2, wa, ba, Wc1, bc1, wc2,
                  mo_o, tr_o):
    cd = cd_in[...]
    radial = jnp.sum(cd * cd, axis=1, keepdims=True)
    m1 = m1_in[...] + radial * w1r[...] + b1[...]
    ms = m1 * jax.nn.sigmoid(m1)
    t = ms @ W2[...] + b2[...]
    m2 = t * jax.nn.sigmoid(t)
    att = jax.nn.sigmoid(jnp.sum(m2 * wa[...], axis=1, keepdims=True)
                         + ba[...][:, 0:1])
    mo = m2 * att
    u = mo @ Wc1[...] + bc1[...]
    cw = u * jax.nn.sigmoid(u)
    cws = jnp.sum(cw * wc2[...], axis=1, keepdims=True)
    one3 = (lax.broadcasted_iota(jnp.int32, (1, 16), 1) == 3).astype(f32)
    mo_o[...] = mo
    tr_o[...] = cd * cws + one3


def _edge_tc(m1, cd, wts):
    be = lambda i: (i, 0)
    w0 = lambda i: (0, 0)
    return pl.pallas_call(
        _edge_tc_body,
        grid=(EE // BE,),
        in_specs=[
            pl.BlockSpec((BE, H), be),
            pl.BlockSpec((BE, 16), be),
            pl.BlockSpec((1, H), w0),
            pl.BlockSpec((1, H), w0),
            pl.BlockSpec((H, H), w0),
            pl.BlockSpec((1, H), w0),
            pl.BlockSpec((1, H), w0),
            pl.BlockSpec((1, H), w0),
            pl.BlockSpec((H, H), w0),
            pl.BlockSpec((1, H), w0),
            pl.BlockSpec((1, H), w0),
        ],
        out_specs=[pl.BlockSpec((BE, H), be), pl.BlockSpec((BE, 16), be)],
        out_shape=[jax.ShapeDtypeStruct((EE, H), f32),
                   jax.ShapeDtypeStruct((EE, 16), f32)],
    )(m1, cd, *wts)


def _node_tc_body(hh, cp, a0, a1, s0, s1, Wn1a, Wn1b, bn1, Wn2, bn2,
                  W1a, W1b, hh_o, cp_o, t1_o, t2_o):
    agg = a0[0] + a1[0]
    s16 = s0[0] + s1[0]
    deg = jnp.clip(s16[:, 3:4], 1.0, None)
    mask3 = (lax.broadcasted_iota(jnp.int32, (1, 16), 1) < 3).astype(f32)
    cpn = cp[...] + (s16 / deg) * mask3
    t = hh[...] @ Wn1a[...] + agg @ Wn1b[...] + bn1[...]
    t = t * jax.nn.sigmoid(t)
    hn = hh[...] + t @ Wn2[...] + bn2[...]
    hh_o[...] = hn
    cp_o[...] = cpn
    t1_o[...] = jnp.concatenate([hn @ W1a[...], cpn], axis=1)
    t2_o[...] = jnp.concatenate([hn @ W1b[...], cpn], axis=1)


def _node_tc(hh, cp, acc, st, w, wn):
    nb = lambda i: (i, 0)
    a0 = lambda i: (0, i, 0)
    a1 = lambda i: (1, i, 0)
    w0 = lambda i: (0, 0)
    return pl.pallas_call(
        _node_tc_body,
        grid=(NN // NB,),
        in_specs=[
            pl.BlockSpec((NB, H), nb),
            pl.BlockSpec((NB, 16), nb),
            pl.BlockSpec((1, NB, H), a0),
            pl.BlockSpec((1, NB, H), a1),
            pl.BlockSpec((1, NB, 16), a0),
            pl.BlockSpec((1, NB, 16), a1),
            pl.BlockSpec((H, H), w0),
            pl.BlockSpec((H, H), w0),
            pl.BlockSpec((1, H), w0),
            pl.BlockSpec((H, H), w0),
            pl.BlockSpec((1, H), w0),
            pl.BlockSpec((H, H), w0),
            pl.BlockSpec((H, H), w0),
        ],
        out_specs=[pl.BlockSpec((NB, H), nb), pl.BlockSpec((NB, 16), nb),
                   pl.BlockSpec((NB, CW), nb), pl.BlockSpec((NB, CW), nb)],
        out_shape=[jax.ShapeDtypeStruct((NN, H), f32),
                   jax.ShapeDtypeStruct((NN, 16), f32),
                   jax.ShapeDtypeStruct((NN, CW), f32),
                   jax.ShapeDtypeStruct((NN, CW), f32)],
    )(hh, cp, acc, acc, st, st, w["Wn1a"], w["Wn1b"], w["bn1"],
      w["Wn2"], w["bn2"], wn["W1a"], wn["W1b"])


def _node_final_body(hh, a0, a1, Wn1a, Wn1b, bn1, Wn2, bn2, Weo, beo, ps_o):
    agg = a0[0] + a1[0]
    t = hh[...] @ Wn1a[...] + agg @ Wn1b[...] + bn1[...]
    t = t * jax.nn.sigmoid(t)
    hn = hh[...] + t @ Wn2[...] + bn2[...]
    eo = hn @ Weo[...] + beo[...]

    @pl.when(pl.program_id(0) == 0)
    def _():
        ps_o[...] = jnp.zeros((1, H), f32)

    ps_o[...] += jnp.sum(eo, axis=0, keepdims=True)


def _node_final(hh, acc, w, Weo, beo):
    nb = lambda i: (i, 0)
    a0 = lambda i: (0, i, 0)
    a1 = lambda i: (1, i, 0)
    w0 = lambda i: (0, 0)
    return pl.pallas_call(
        _node_final_body,
        grid=(NN // NB,),
        in_specs=[
            pl.BlockSpec((NB, H), nb),
            pl.BlockSpec((1, NB, H), a0),
            pl.BlockSpec((1, NB, H), a1),
            pl.BlockSpec((H, H), w0),
            pl.BlockSpec((H, H), w0),
            pl.BlockSpec((1, H), w0),
            pl.BlockSpec((H, H), w0),
            pl.BlockSpec((1, H), w0),
            pl.BlockSpec((H, H), w0),
            pl.BlockSpec((1, H), w0),
        ],
        out_specs=pl.BlockSpec((1, H), w0),
        out_shape=jax.ShapeDtypeStruct((1, H), f32),
    )(hh, acc, acc, w["Wn1a"], w["Wn1b"], w["bn1"], w["Wn2"], w["bn2"],
      Weo, beo)


def _emb_body(h_in, xp, We, be_, W1a, W1b, hh_o, t1_o, t2_o):
    hh = h_in[...] @ We[...] + be_[...]
    hh_o[...] = hh
    t1_o[...] = jnp.concatenate([hh @ W1a[...], xp[...]], axis=1)
    t2_o[...] = jnp.concatenate([hh @ W1b[...], xp[...]], axis=1)


def _emb(h, xp, We, be_, W1a, W1b):
    nb = lambda i: (i, 0)
    w0 = lambda i: (0, 0)
    return pl.pallas_call(
        _emb_body,
        grid=(NN // NB,),
        in_specs=[
            pl.BlockSpec((NB, H), nb),
            pl.BlockSpec((NB, 16), nb),
            pl.BlockSpec((H, H), w0),
            pl.BlockSpec((1, H), w0),
            pl.BlockSpec((H, H), w0),
            pl.BlockSpec((H, H), w0),
        ],
        out_specs=[pl.BlockSpec((NB, H), nb), pl.BlockSpec((NB, CW), nb),
                   pl.BlockSpec((NB, CW), nb)],
        out_shape=[jax.ShapeDtypeStruct((NN, H), f32),
                   jax.ShapeDtypeStruct((NN, CW), f32),
                   jax.ShapeDtypeStruct((NN, CW), f32)],
    )(h, xp, We, be_, W1a, W1b)


def _head_body(ps, Wm1, bm1, Wm2, bm2, o_ref):
    hm = ps[...] * (1.0 / NN)
    z = hm @ Wm1[...] + bm1[...]
    z = jnp.maximum(z, 0.0)
    o_ref[...] = z @ Wm2[...] + bm2[...]


def _head(ps, Wm1, bm1, Wm2, bm2):
    w0 = lambda i: (0, 0)
    return pl.pallas_call(
        _head_body,
        grid=(1,),
        in_specs=[
            pl.BlockSpec((1, H), w0),
            pl.BlockSpec((H, H), w0),
            pl.BlockSpec((1, H), w0),
            pl.BlockSpec((H, H), w0),
            pl.BlockSpec((1, H), w0),
        ],
        out_specs=pl.BlockSpec((1, H), w0),
        out_shape=jax.ShapeDtypeStruct((1, H), f32),
    )(ps, Wm1, bm1, Wm2, bm2)


def _layer_views(p):
    W1 = p["edge1"]["W"]
    return dict(
        W1a=W1[:H],
        W1b=W1[H:2 * H],
        ew=(W1[2 * H:2 * H + 1],
            p["edge1"]["b"].reshape(1, H),
            p["edge2"]["W"],
            p["edge2"]["b"].reshape(1, H),
            p["att"]["W"].T,
            jnp.broadcast_to(p["att"]["b"].reshape(1, 1), (1, H)),
            p["coord1"]["W"],
            p["coord1"]["b"].reshape(1, H),
            p["coord2"]["W"].T),
        Wn1a=p["node1"]["W"][:H],
        Wn1b=p["node1"]["W"][H:],
        bn1=p["node1"]["b"].reshape(1, H),
        Wn2=p["node2"]["W"],
        bn2=p["node2"]["b"].reshape(1, H),
    )


def kernel(h, x, edges, params):
    rowr = edges[0].reshape(NW * NSTEP, CH)
    colr = edges[1].reshape(NW * NSTEP, CH)
    xp = jnp.pad(x, ((0, 0), (0, 13)))
    z128 = jnp.zeros((NPAD, H), f32)
    z16 = jnp.zeros((NPAD, 16), f32)
    lws = [_layer_views(p) for p in params["layers"]]
    hh, t1, t2 = _emb(h, xp, params["emb_in"]["W"],
                      params["emb_in"]["b"].reshape(1, H),
                      lws[0]["W1a"], lws[0]["W1b"])
    cp = xp
    ps = None
    for l in range(4):
        m1, cd = _sc_gather(t1, t2, rowr, colr)
        mo, tr = _edge_tc(m1, cd, lws[l]["ew"])
        acc, st = _sc_scatter(mo, tr, rowr, z128, z16)
        if l < 3:
            hh, cp, t1, t2 = _node_tc(hh, cp, acc, st, lws[l], lws[l + 1])
        else:
            ps = _node_final(hh, acc, lws[l], params["emb_out"]["W"],
                             params["emb_out"]["b"].reshape(1, H))
    o = _head(ps, params["mlp1"]["W"], params["mlp1"]["b"].reshape(1, H),
              jnp.pad(params["mlp2"]["W"], ((0, 0), (0, H - 20))),
              jnp.pad(params["mlp2"]["b"], (0, H - 20)).reshape(1, H))
    return o[0, :20]


# speedup vs baseline: 3.3511x; 1.0004x over previous
import jax
import jax.numpy as jnp
from jax import lax
from jax.experimental import pallas as pl
from jax.experimental.pallas import tpu as pltpu
from jax.experimental.pallas import tpu_sc as plsc

NN = 10000
EE = 320000
H = 128
CW = 144
NC, NS = 2, 16
NW = NC * NS
EW = EE // NW
CH = 80
NSTEP = EW // CH
NPAD = 10240
RPT = NPAD // NS
NB = 2000
BE = 5000

f32 = jnp.float32


def _mesh():
    return plsc.VectorSubcoreMesh(core_axis_name="c", subcore_axis_name="s",
                                  num_cores=NC, num_subcores=NS)


_SC_PARAMS = pltpu.CompilerParams(use_tc_tiling_on_sc=False)


def _sc_gather_body(t1_hbm, t2_hbm, rowr_hbm, colr_hbm, m1_hbm, cd_hbm,
                    ridx, cidx, b10, b20, ob0, cdb0, b11, b21, ob1, cdb1,
                    gs0, gs1, ws0, ws1, wc0, wc1):
    cid = lax.axis_index("c")
    sid = lax.axis_index("s")
    wid = sid * NC + cid
    cb = wid * NSTEP
    pltpu.sync_copy(rowr_hbm.at[pl.ds(cb, NSTEP)], ridx)
    pltpu.sync_copy(colr_hbm.at[pl.ds(cb, NSTEP)], cidx)
    bufs = ((b10, b20, ob0, cdb0), (b11, b21, ob1, cdb1))
    gsems = (gs0, gs1)
    wsems = (ws0, ws1)
    wcsems = (wc0, wc1)

    def fire(k, s):
        b1, b2, _, _ = bufs[s]
        pltpu.async_copy(t1_hbm.at[ridx.at[k]], b1, gsems[s])
        pltpu.async_copy(t2_hbm.at[cidx.at[k]], b2, gsems[s])

    def waitg(s):
        b1, b2, _, _ = bufs[s]
        pltpu.make_async_copy(t1_hbm.at[ridx.at[0]], b1, gsems[s]).wait()
        pltpu.make_async_copy(t1_hbm.at[ridx.at[0]], b2, gsems[s]).wait()

    def compute(s):
        b1, b2, ob, cdb = bufs[s]

        def rowf(r, carry):
            for c in range(8):
                sl = pl.ds(c * 16, 16)
                ob[r, sl] = b1[r, sl] + b2[r, sl]
            sc = pl.ds(H, 16)
            cdb[r, pl.ds(0, 16)] = b1[r, sc] - b2[r, sc]
            return carry

        lax.fori_loop(0, CH, rowf, 0)

    def firew(k, s):
        _, _, ob, cdb = bufs[s]
        pltpu.async_copy(ob, m1_hbm.at[pl.ds((cb + k) * CH, CH)], wsems[s])
        pltpu.async_copy(cdb, cd_hbm.at[pl.ds((cb + k) * CH, CH)], wcsems[s])

    def waitw(s):
        _, _, ob, cdb = bufs[s]
        pltpu.make_async_copy(ob, m1_hbm.at[pl.ds(cb * CH, CH)],
                              wsems[s]).wait()
        pltpu.make_async_copy(cdb, cd_hbm.at[pl.ds(cb * CH, CH)],
                              wcsems[s]).wait()

    fire(0, 0)
    fire(1, 1)
    waitg(0)
    compute(0)
    firew(0, 0)

    def body(j, carry):
        k1 = 2 * j + 1
        waitw(0)
        fire(k1 + 1, 0)
        waitg(1)
        compute(1)
        firew(k1, 1)
        k2 = k1 + 1
        waitw(1)
        fire(k2 + 1, 1)
        waitg(0)
        compute(0)
        firew(k2, 0)
        return carry

    lax.fori_loop(0, (NSTEP - 3) // 2, body, 0)
    waitw(0)
    fire(NSTEP - 1, 0)
    waitg(1)
    compute(1)
    firew(NSTEP - 2, 1)
    waitg(0)
    compute(0)
    firew(NSTEP - 1, 0)
    waitw(1)
    waitw(0)


def _sc_gather(t1, t2, rowr, colr):
    f = pl.kernel(
        _sc_gather_body,
        out_type=(
            jax.ShapeDtypeStruct((EE, H), f32),
            jax.ShapeDtypeStruct((EE, 16), f32),
        ),
        mesh=_mesh(),
        scratch_types=(
            pltpu.VMEM((NSTEP, CH), jnp.int32),
            pltpu.VMEM((NSTEP, CH), jnp.int32),
            pltpu.VMEM((CH, CW), f32),
            pltpu.VMEM((CH, CW), f32),
            pltpu.VMEM((CH, H), f32),
            pltpu.VMEM((CH, 16), f32),
            pltpu.VMEM((CH, CW), f32),
            pltpu.VMEM((CH, CW), f32),
            pltpu.VMEM((CH, H), f32),
            pltpu.VMEM((CH, 16), f32),
            pltpu.SemaphoreType.DMA,
            pltpu.SemaphoreType.DMA,
            pltpu.SemaphoreType.DMA,
            pltpu.SemaphoreType.DMA,
            pltpu.SemaphoreType.DMA,
            pltpu.SemaphoreType.DMA,
        ),
        compiler_params=_SC_PARAMS,
    )
    return f(t1, t2, rowr, colr)


def _sc_scatter_body(mo_hbm, tr_hbm, rowr_hbm, z128, z16, acc_out, s_out,
                     ridx, mb0, tb0, mb1, tb1, ls0, ls1, acc_sh, s_sh):
    cid = lax.axis_index("c")
    sid = lax.axis_index("s")
    wid = sid * NC + cid
    rb = sid * RPT
    pltpu.sync_copy(z128.at[pl.ds(rb, RPT)], acc_sh.at[pl.ds(rb, RPT)])
    pltpu.sync_copy(z16.at[pl.ds(rb, RPT)], s_sh.at[pl.ds(rb, RPT)])
    cb = wid * NSTEP
    pltpu.sync_copy(rowr_hbm.at[pl.ds(cb, NSTEP)], ridx)
    plsc.subcore_barrier()
    bufs = ((mb0, tb0), (mb1, tb1))
    lsems = (ls0, ls1)

    def load(k, s):
        mb, tb = bufs[s]
        pltpu.async_copy(mo_hbm.at[pl.ds((cb + k) * CH, CH)], mb, lsems[s])
        pltpu.async_copy(tr_hbm.at[pl.ds((cb + k) * CH, CH)], tb, lsems[s])

    def waitl(s):
        mb, tb = bufs[s]
        pltpu.make_async_copy(mo_hbm.at[pl.ds(cb * CH, CH)], mb,
                              lsems[s]).wait()
        pltpu.make_async_copy(tr_hbm.at[pl.ds(cb * CH, CH)], tb,
                              lsems[s]).wait()

    def scat(k, s):
        mb, tb = bufs[s]
        pltpu.sync_copy(mb, acc_sh.at[ridx.at[k]], add=True)
        pltpu.sync_copy(tb, s_sh.at[ridx.at[k]], add=True)

    load(0, 0)
    load(1, 1)
    waitl(0)
    scat(0, 0)
    load(2, 0)

    def body(j, carry):
        k1 = 2 * j + 1
        waitl(1)
        scat(k1, 1)
        load(k1 + 2, 1)
        k2 = k1 + 1
        waitl(0)
        scat(k2, 0)
        load(k2 + 2, 0)
        return carry

    lax.fori_loop(0, (NSTEP - 3) // 2, body, 0)
    waitl(1)
    scat(NSTEP - 2, 1)
    waitl(0)
    scat(NSTEP - 1, 0)
    plsc.subcore_barrier()
    pltpu.sync_copy(acc_sh.at[pl.ds(rb, RPT)], acc_out.at[cid, pl.ds(rb, RPT)])
    pltpu.sync_copy(s_sh.at[pl.ds(rb, RPT)], s_out.at[cid, pl.ds(rb, RPT)])


def _sc_scatter(mo, tr, rowr, z128, z16):
    f = pl.kernel(
        _sc_scatter_body,
        out_type=(
            jax.ShapeDtypeStruct((NC, NPAD, H), f32),
            jax.ShapeDtypeStruct((NC, NPAD, 16), f32),
        ),
        mesh=_mesh(),
        scratch_types=(
            pltpu.VMEM((NSTEP, CH), jnp.int32),
            pltpu.VMEM((CH, H), f32),
            pltpu.VMEM((CH, 16), f32),
            pltpu.VMEM((CH, H), f32),
            pltpu.VMEM((CH, 16), f32),
            pltpu.SemaphoreType.DMA,
            pltpu.SemaphoreType.DMA,
            pltpu.VMEM_SHARED((NPAD, H), f32),
            pltpu.VMEM_SHARED((NPAD, 16), f32),
        ),
        compiler_params=_SC_PARAMS,
    )
    return f(mo, tr, rowr, z128, z16)


def _edge_tc_body(m1_in, cd_in, w1r, b1, W2, b2, wa, ba, Wc1, bc1, wc2,
                  mo_o, tr_o):
    cd = cd_in[...]
    radial = jnp.sum(cd * cd, axis=1, keepdims=True)
    m1 = m1_in[...] + radial * w1r[...] + b1[...]
    ms = m1 * jax.nn.sigmoid(m1)
    t = ms @ W2[...] + b2[...]
    m2 = t * jax.nn.sigmoid(t)
    att = jax.nn.sigmoid(jnp.sum(m2 * wa[...], axis=1, keepdims=True)
                         + ba[...][:, 0:1])
    mo = m2 * att
    u = mo @ Wc1[...] + bc1[...]
    cw = u * jax.nn.sigmoid(u)
    cws = jnp.sum(cw * wc2[...], axis=1, keepdims=True)
    one3 = (lax.broadcasted_iota(jnp.int32, (1, 16), 1) == 3).astype(f32)
    mo_o[...] = mo
    tr_o[...] = cd * cws + one3


def _edge_tc(m1, cd, wts):
    be = lambda i: (i, 0)
    w0 = lambda i: (0, 0)
    return pl.pallas_call(
        _edge_tc_body,
        grid=(EE // BE,),
        in_specs=[
            pl.BlockSpec((BE, H), be),
            pl.BlockSpec((BE, 16), be),
            pl.BlockSpec((1, H), w0),
            pl.BlockSpec((1, H), w0),
            pl.BlockSpec((H, H), w0),
            pl.BlockSpec((1, H), w0),
            pl.BlockSpec((1, H), w0),
            pl.BlockSpec((1, H), w0),
            pl.BlockSpec((H, H), w0),
            pl.BlockSpec((1, H), w0),
            pl.BlockSpec((1, H), w0),
        ],
        out_specs=[pl.BlockSpec((BE, H), be), pl.BlockSpec((BE, 16), be)],
        out_shape=[jax.ShapeDtypeStruct((EE, H), f32),
                   jax.ShapeDtypeStruct((EE, 16), f32)],
    )(m1, cd, *wts)


def _node_tc_body(hh, cp, a0, a1, s0, s1, Wn1a, Wn1b, bn1, Wn2, bn2,
                  W1a, W1b, hh_o, cp_o, t1_o, t2_o):
    agg = a0[0] + a1[0]
    s16 = s0[0] + s1[0]
    deg = jnp.clip(s16[:, 3:4], 1.0, None)
    mask3 = (lax.broadcasted_iota(jnp.int32, (1, 16), 1) < 3).astype(f32)
    cpn = cp[...] + (s16 / deg) * mask3
    t = hh[...] @ Wn1a[...] + agg @ Wn1b[...] + bn1[...]
    t = t * jax.nn.sigmoid(t)
    hn = hh[...] + t @ Wn2[...] + bn2[...]
    hh_o[...] = hn
    cp_o[...] = cpn
    t1_o[...] = jnp.concatenate([hn @ W1a[...], cpn], axis=1)
    t2_o[...] = jnp.concatenate([hn @ W1b[...], cpn], axis=1)


def _node_tc(hh, cp, acc, st, w, wn):
    nb = lambda i: (i, 0)
    a0 = lambda i: (0, i, 0)
    a1 = lambda i: (1, i, 0)
    w0 = lambda i: (0, 0)
    return pl.pallas_call(
        _node_tc_body,
        grid=(NN // NB,),
        in_specs=[
            pl.BlockSpec((NB, H), nb),
            pl.BlockSpec((NB, 16), nb),
            pl.BlockSpec((1, NB, H), a0),
            pl.BlockSpec((1, NB, H), a1),
            pl.BlockSpec((1, NB, 16), a0),
            pl.BlockSpec((1, NB, 16), a1),
            pl.BlockSpec((H, H), w0),
            pl.BlockSpec((H, H), w0),
            pl.BlockSpec((1, H), w0),
            pl.BlockSpec((H, H), w0),
            pl.BlockSpec((1, H), w0),
            pl.BlockSpec((H, H), w0),
            pl.BlockSpec((H, H), w0),
        ],
        out_specs=[pl.BlockSpec((NB, H), nb), pl.BlockSpec((NB, 16), nb),
                   pl.BlockSpec((NB, CW), nb), pl.BlockSpec((NB, CW), nb)],
        out_shape=[jax.ShapeDtypeStruct((NN, H), f32),
                   jax.ShapeDtypeStruct((NN, 16), f32),
                   jax.ShapeDtypeStruct((NN, CW), f32),
                   jax.ShapeDtypeStruct((NN, CW), f32)],
    )(hh, cp, acc, acc, st, st, w["Wn1a"], w["Wn1b"], w["bn1"],
      w["Wn2"], w["bn2"], wn["W1a"], wn["W1b"])


def _node_final_body(hh, a0, a1, Wn1a, Wn1b, bn1, Wn2, bn2, Weo, beo, ps_o):
    agg = a0[0] + a1[0]
    t = hh[...] @ Wn1a[...] + agg @ Wn1b[...] + bn1[...]
    t = t * jax.nn.sigmoid(t)
    hn = hh[...] + t @ Wn2[...] + bn2[...]
    eo = hn @ Weo[...] + beo[...]

    @pl.when(pl.program_id(0) == 0)
    def _():
        ps_o[...] = jnp.zeros((1, H), f32)

    ps_o[...] += jnp.sum(eo, axis=0, keepdims=True)


def _node_final(hh, acc, w, Weo, beo):
    nb = lambda i: (i, 0)
    a0 = lambda i: (0, i, 0)
    a1 = lambda i: (1, i, 0)
    w0 = lambda i: (0, 0)
    return pl.pallas_call(
        _node_final_body,
        grid=(NN // NB,),
        in_specs=[
            pl.BlockSpec((NB, H), nb),
            pl.BlockSpec((1, NB, H), a0),
            pl.BlockSpec((1, NB, H), a1),
            pl.BlockSpec((H, H), w0),
            pl.BlockSpec((H, H), w0),
            pl.BlockSpec((1, H), w0),
            pl.BlockSpec((H, H), w0),
            pl.BlockSpec((1, H), w0),
            pl.BlockSpec((H, H), w0),
            pl.BlockSpec((1, H), w0),
        ],
        out_specs=pl.BlockSpec((1, H), w0),
        out_shape=jax.ShapeDtypeStruct((1, H), f32),
    )(hh, acc, acc, w["Wn1a"], w["Wn1b"], w["bn1"], w["Wn2"], w["bn2"],
      Weo, beo)


def _emb_body(h_in, xp, We, be_, W1a, W1b, hh_o, t1_o, t2_o):
    hh = h_in[...] @ We[...] + be_[...]
    hh_o[...] = hh
    t1_o[...] = jnp.concatenate([hh @ W1a[...], xp[...]], axis=1)
    t2_o[...] = jnp.concatenate([hh @ W1b[...], xp[...]], axis=1)


def _emb(h, xp, We, be_, W1a, W1b):
    nb = lambda i: (i, 0)
    w0 = lambda i: (0, 0)
    return pl.pallas_call(
        _emb_body,
        grid=(NN // NB,),
        in_specs=[
            pl.BlockSpec((NB, H), nb),
            pl.BlockSpec((NB, 16), nb),
            pl.BlockSpec((H, H), w0),
            pl.BlockSpec((1, H), w0),
            pl.BlockSpec((H, H), w0),
            pl.BlockSpec((H, H), w0),
        ],
        out_specs=[pl.BlockSpec((NB, H), nb), pl.BlockSpec((NB, CW), nb),
                   pl.BlockSpec((NB, CW), nb)],
        out_shape=[jax.ShapeDtypeStruct((NN, H), f32),
                   jax.ShapeDtypeStruct((NN, CW), f32),
                   jax.ShapeDtypeStruct((NN, CW), f32)],
    )(h, xp, We, be_, W1a, W1b)


def _head_body(ps, Wm1, bm1, Wm2, bm2, o_ref):
    hm = ps[...] * (1.0 / NN)
    z = hm @ Wm1[...] + bm1[...]
    z = jnp.maximum(z, 0.0)
    o_ref[...] = z @ Wm2[...] + bm2[...]


def _head(ps, Wm1, bm1, Wm2, bm2):
    w0 = lambda i: (0, 0)
    return pl.pallas_call(
        _head_body,
        grid=(1,),
        in_specs=[
            pl.BlockSpec((1, H), w0),
            pl.BlockSpec((H, H), w0),
            pl.BlockSpec((1, H), w0),
            pl.BlockSpec((H, H), w0),
            pl.BlockSpec((1, H), w0),
        ],
        out_specs=pl.BlockSpec((1, H), w0),
        out_shape=jax.ShapeDtypeStruct((1, H), f32),
    )(ps, Wm1, bm1, Wm2, bm2)


def _layer_views(p):
    W1 = p["edge1"]["W"]
    return dict(
        W1a=W1[:H],
        W1b=W1[H:2 * H],
        ew=(W1[2 * H:2 * H + 1],
            p["edge1"]["b"].reshape(1, H),
            p["edge2"]["W"],
            p["edge2"]["b"].reshape(1, H),
            p["att"]["W"].T,
            jnp.broadcast_to(p["att"]["b"].reshape(1, 1), (1, H)),
            p["coord1"]["W"],
            p["coord1"]["b"].reshape(1, H),
            p["coord2"]["W"].T),
        Wn1a=p["node1"]["W"][:H],
        Wn1b=p["node1"]["W"][H:],
        bn1=p["node1"]["b"].reshape(1, H),
        Wn2=p["node2"]["W"],
        bn2=p["node2"]["b"].reshape(1, H),
    )


def kernel(h, x, edges, params):
    rowr = edges[0].reshape(NW * NSTEP, CH)
    colr = edges[1].reshape(NW * NSTEP, CH)
    xp = jnp.pad(x, ((0, 0), (0, 13)))
    z128 = jnp.zeros((NPAD, H), f32)
    z16 = jnp.zeros((NPAD, 16), f32)
    lws = [_layer_views(p) for p in params["layers"]]
    hh, t1, t2 = _emb(h, xp, params["emb_in"]["W"],
                      params["emb_in"]["b"].reshape(1, H),
                      lws[0]["W1a"], lws[0]["W1b"])
    cp = xp
    ps = None
    for l in range(4):
        m1, cd = _sc_gather(t1, t2, rowr, colr)
        mo, tr = _edge_tc(m1, cd, lws[l]["ew"])
        acc, st = _sc_scatter(mo, tr, rowr, z128, z16)
        if l < 3:
            hh, cp, t1, t2 = _node_tc(hh, cp, acc, st, lws[l], lws[l + 1])
        else:
            ps = _node_final(hh, acc, lws[l], params["emb_out"]["W"],
                             params["emb_out"]["b"].reshape(1, H))
    o = _head(ps, params["mlp1"]["W"], params["mlp1"]["b"].reshape(1, H),
              jnp.pad(params["mlp2"]["W"], ((0, 0), (0, H - 20))),
              jnp.pad(params["mlp2"]["b"], (0, H - 20)).reshape(1, H))
    return o[0, :20]


# speedup vs baseline: 4.3758x; 1.3058x over previous
import functools

import jax
import jax.numpy as jnp
from jax import lax
from jax.experimental import pallas as pl
from jax.experimental.pallas import tpu as pltpu
from jax.experimental.pallas import tpu_sc as plsc

NN = 10000
EE = 320000
EA = 192000
EB = 128000
H = 128
CW = 144
NC, NS = 2, 16
NW = NC * NS
CH = 80
NPAD = 10240
RPT = NPAD // NS
NB = 2000
BE = 4000

f32 = jnp.float32
bf16 = jnp.bfloat16


def _mesh():
    return plsc.VectorSubcoreMesh(core_axis_name="c", subcore_axis_name="s",
                                  num_cores=NC, num_subcores=NS)


_SC_PARAMS = pltpu.CompilerParams(use_tc_tiling_on_sc=False)


def _gather_body(nstep, t1_hbm, t2_hbm, rowr_hbm, colr_hbm, m1_hbm, cd_hbm,
                    ridx, cidx, b10, b20, ob0, cdb0, b11, b21, ob1, cdb1,
                    gs0, gs1, ws0, ws1):
    cid = lax.axis_index("c")
    sid = lax.axis_index("s")
    wid = sid * NC + cid
    cb = wid * nstep
    pltpu.sync_copy(rowr_hbm.at[pl.ds(cb, nstep)], ridx)
    pltpu.sync_copy(colr_hbm.at[pl.ds(cb, nstep)], cidx)
    bufs = ((b10, b20, ob0, cdb0), (b11, b21, ob1, cdb1))
    gsems = (gs0, gs1)
    wsems = (ws0, ws1)

    def fire(k, s):
        b1, b2, _, _ = bufs[s]
        pltpu.async_copy(t1_hbm.at[ridx.at[k]], b1, gsems[s])
        pltpu.async_copy(t2_hbm.at[cidx.at[k]], b2, gsems[s])

    def waitg(s):
        b1, b2, _, _ = bufs[s]
        pltpu.make_async_copy(t1_hbm.at[ridx.at[0]], b1, gsems[s]).wait()
        pltpu.make_async_copy(t1_hbm.at[ridx.at[0]], b2, gsems[s]).wait()

    def compute(s):
        b1, b2, ob, cdb = bufs[s]

        def rowf(r, carry):
            for c in range(8):
                sl = pl.ds(c * 16, 16)
                ob[r, sl] = b1[r, sl] + b2[r, sl]
            sc = pl.ds(H, 16)
            cdb[r, pl.ds(0, 16)] = b1[r, sc] - b2[r, sc]
            return carry

        lax.fori_loop(0, CH, rowf, 0)

    def firew(k, s):
        _, _, ob, cdb = bufs[s]
        pltpu.async_copy(ob, m1_hbm.at[pl.ds((cb + k) * CH, CH)], wsems[s])
        pltpu.async_copy(cdb, cd_hbm.at[pl.ds((cb + k) * CH, CH)], wsems[s])

    def waitw(s):
        _, _, ob, cdb = bufs[s]
        pltpu.make_async_copy(ob, m1_hbm.at[pl.ds(cb * CH, CH)],
                              wsems[s]).wait()
        pltpu.make_async_copy(cdb, cd_hbm.at[pl.ds(cb * CH, CH)],
                              wsems[s]).wait()

    fire(0, 0)
    fire(1, 1)
    waitg(0)
    compute(0)
    firew(0, 0)

    def body(j, carry):
        k1 = 2 * j + 1
        waitw(0)
        fire(k1 + 1, 0)
        waitg(1)
        compute(1)
        firew(k1, 1)
        k2 = k1 + 1
        waitw(1)
        fire(k2 + 1, 1)
        waitg(0)
        compute(0)
        firew(k2, 0)
        return carry

    P = (nstep - 3) // 2
    lax.fori_loop(0, P, body, 0)
    nxt = 2 * P + 1
    while nxt < nstep:
        if nxt + 1 < nstep:
            sx = (nxt + 1) % 2
            waitw(sx)
            fire(nxt + 1, sx)
        waitg(nxt % 2)
        compute(nxt % 2)
        firew(nxt, nxt % 2)
        nxt += 1
    waitw((nstep - 1) % 2)
    waitw((nstep - 2) % 2)


def _sc_gather(t1, t2, rowr, colr, ecount):
    nstep = ecount // NW // CH
    f = pl.kernel(
        functools.partial(_gather_body, nstep),
        out_type=(
            jax.ShapeDtypeStruct((ecount, H), f32),
            jax.ShapeDtypeStruct((ecount, 16), f32),
        ),
        mesh=_mesh(),
        scratch_types=(
            pltpu.VMEM((nstep, CH), jnp.int32),
            pltpu.VMEM((nstep, CH), jnp.int32),
            pltpu.VMEM((CH, CW), f32),
            pltpu.VMEM((CH, CW), f32),
            pltpu.VMEM((CH, H), f32),
            pltpu.VMEM((CH, 16), f32),
            pltpu.VMEM((CH, CW), f32),
            pltpu.VMEM((CH, CW), f32),
            pltpu.VMEM((CH, H), f32),
            pltpu.VMEM((CH, 16), f32),
            pltpu.SemaphoreType.DMA,
            pltpu.SemaphoreType.DMA,
            pltpu.SemaphoreType.DMA,
            pltpu.SemaphoreType.DMA,
        ),
        compiler_params=_SC_PARAMS,
    )
    return f(t1, t2, rowr, colr)


def _scatter_body(nstep, mo_hbm, tr_hbm, rowr_hbm, z128, z16, acc_out, s_out,
                     ridx, mb0, tb0, mb1, tb1, ls0, ls1, acc_sh, s_sh):
    cid = lax.axis_index("c")
    sid = lax.axis_index("s")
    wid = sid * NC + cid
    rb = sid * RPT
    pltpu.sync_copy(z128.at[pl.ds(rb, RPT)], acc_sh.at[pl.ds(rb, RPT)])
    pltpu.sync_copy(z16.at[pl.ds(rb, RPT)], s_sh.at[pl.ds(rb, RPT)])
    cb = wid * nstep
    pltpu.sync_copy(rowr_hbm.at[pl.ds(cb, nstep)], ridx)
    plsc.subcore_barrier()
    bufs = ((mb0, tb0), (mb1, tb1))
    lsems = (ls0, ls1)

    def load(k, s):
        mb, tb = bufs[s]
        pltpu.async_copy(mo_hbm.at[pl.ds((cb + k) * CH, CH)], mb, lsems[s])
        pltpu.async_copy(tr_hbm.at[pl.ds((cb + k) * CH, CH)], tb, lsems[s])

    def waitl(s):
        mb, tb = bufs[s]
        pltpu.make_async_copy(mo_hbm.at[pl.ds(cb * CH, CH)], mb,
                              lsems[s]).wait()
        pltpu.make_async_copy(tr_hbm.at[pl.ds(cb * CH, CH)], tb,
                              lsems[s]).wait()

    def scat(k, s):
        mb, tb = bufs[s]
        pltpu.sync_copy(mb, acc_sh.at[ridx.at[k]], add=True)
        pltpu.sync_copy(tb, s_sh.at[ridx.at[k]], add=True)

    load(0, 0)
    load(1, 1)
    waitl(0)
    scat(0, 0)
    load(2, 0)

    def body(j, carry):
        k1 = 2 * j + 1
        waitl(1)
        scat(k1, 1)
        load(k1 + 2, 1)
        k2 = k1 + 1
        waitl(0)
        scat(k2, 0)
        load(k2 + 2, 0)
        return carry

    P = (nstep - 3) // 2
    lax.fori_loop(0, P, body, 0)
    nxt = 2 * P + 1
    while nxt < nstep:
        waitl(nxt % 2)
        scat(nxt, nxt % 2)
        if nxt + 2 < nstep:
            load(nxt + 2, nxt % 2)
        nxt += 1
    plsc.subcore_barrier()
    pltpu.sync_copy(acc_sh.at[pl.ds(rb, RPT)], acc_out.at[cid, pl.ds(rb, RPT)])
    pltpu.sync_copy(s_sh.at[pl.ds(rb, RPT)], s_out.at[cid, pl.ds(rb, RPT)])


def _sc_scatter(mo, tr, rowr, z128, z16, ecount):
    nstep = ecount // NW // CH
    f = pl.kernel(
        functools.partial(_scatter_body, nstep),
        out_type=(
            jax.ShapeDtypeStruct((NC, NPAD, H), f32),
            jax.ShapeDtypeStruct((NC, NPAD, 16), f32),
        ),
        mesh=_mesh(),
        scratch_types=(
            pltpu.VMEM((nstep, CH), jnp.int32),
            pltpu.VMEM((CH, H), f32),
            pltpu.VMEM((CH, 16), f32),
            pltpu.VMEM((CH, H), f32),
            pltpu.VMEM((CH, 16), f32),
            pltpu.SemaphoreType.DMA,
            pltpu.SemaphoreType.DMA,
            pltpu.VMEM_SHARED((NPAD, H), f32),
            pltpu.VMEM_SHARED((NPAD, 16), f32),
        ),
        compiler_params=_SC_PARAMS,
    )
    return f(mo, tr, rowr, z128, z16)


def _edge_tc_body(m1_in, cd_in, w1r, b1, W2, b2, wa, ba, Wc1, bc1, wc2,
                  mo_o, tr_o):
    cd = cd_in[...]
    radial = jnp.sum(cd * cd, axis=1, keepdims=True)
    m1 = m1_in[...] + radial * w1r[...] + b1[...]
    ms = m1 * jax.nn.sigmoid(m1)
    t = jnp.dot(ms.astype(bf16), W2[...],
                preferred_element_type=f32) + b2[...]
    m2 = t * jax.nn.sigmoid(t)
    att = jax.nn.sigmoid(jnp.sum(m2 * wa[...], axis=1, keepdims=True)
                         + ba[...][:, 0:1])
    mo = m2 * att
    u = jnp.dot(mo.astype(bf16), Wc1[...],
                preferred_element_type=f32) + bc1[...]
    cw = u * jax.nn.sigmoid(u)
    cws = jnp.sum(cw * wc2[...], axis=1, keepdims=True)
    one3 = (lax.broadcasted_iota(jnp.int32, (1, 16), 1) == 3).astype(f32)
    mo_o[...] = mo
    tr_o[...] = cd * cws + one3


def _edge_tc(m1, cd, wts, ecount):
    be = lambda i: (i, 0)
    w0 = lambda i: (0, 0)
    return pl.pallas_call(
        _edge_tc_body,
        grid=(ecount // BE,),
        in_specs=[
            pl.BlockSpec((BE, H), be),
            pl.BlockSpec((BE, 16), be),
            pl.BlockSpec((1, H), w0),
            pl.BlockSpec((1, H), w0),
            pl.BlockSpec((H, H), w0),
            pl.BlockSpec((1, H), w0),
            pl.BlockSpec((1, H), w0),
            pl.BlockSpec((1, H), w0),
            pl.BlockSpec((H, H), w0),
            pl.BlockSpec((1, H), w0),
            pl.BlockSpec((1, H), w0),
        ],
        out_specs=[pl.BlockSpec((BE, H), be), pl.BlockSpec((BE, 16), be)],
        out_shape=[jax.ShapeDtypeStruct((ecount, H), f32),
                   jax.ShapeDtypeStruct((ecount, 16), f32)],
    )(m1, cd, *wts)


def _node_tc_body(hh, cp, a0, a1, a2, a3, s0, s1, s2, s3,
                  Wn1a, Wn1b, bn1, Wn2, bn2,
                  W1a, W1b, hh_o, cp_o, t1_o, t2_o):
    agg = (a0[0] + a1[0]) + (a2[0] + a3[0])
    s16 = (s0[0] + s1[0]) + (s2[0] + s3[0])
    deg = jnp.clip(s16[:, 3:4], 1.0, None)
    mask3 = (lax.broadcasted_iota(jnp.int32, (1, 16), 1) < 3).astype(f32)
    cpn = cp[...] + (s16 / deg) * mask3
    t = hh[...] @ Wn1a[...] + agg @ Wn1b[...] + bn1[...]
    t = t * jax.nn.sigmoid(t)
    hn = hh[...] + t @ Wn2[...] + bn2[...]
    hh_o[...] = hn
    cp_o[...] = cpn
    t1_o[...] = jnp.concatenate([hn @ W1a[...], cpn], axis=1)
    t2_o[...] = jnp.concatenate([hn @ W1b[...], cpn], axis=1)


def _node_tc(hh, cp, acc, st, accB, stB, w, wn):
    nb = lambda i: (i, 0)
    a0 = lambda i: (0, i, 0)
    a1 = lambda i: (1, i, 0)
    w0 = lambda i: (0, 0)
    return pl.pallas_call(
        _node_tc_body,
        grid=(NN // NB,),
        in_specs=[
            pl.BlockSpec((NB, H), nb),
            pl.BlockSpec((NB, 16), nb),
            pl.BlockSpec((1, NB, H), a0),
            pl.BlockSpec((1, NB, H), a1),
            pl.BlockSpec((1, NB, H), a0),
            pl.BlockSpec((1, NB, H), a1),
            pl.BlockSpec((1, NB, 16), a0),
            pl.BlockSpec((1, NB, 16), a1),
            pl.BlockSpec((1, NB, 16), a0),
            pl.BlockSpec((1, NB, 16), a1),
            pl.BlockSpec((H, H), w0),
            pl.BlockSpec((H, H), w0),
            pl.BlockSpec((1, H), w0),
            pl.BlockSpec((H, H), w0),
            pl.BlockSpec((1, H), w0),
            pl.BlockSpec((H, H), w0),
            pl.BlockSpec((H, H), w0),
        ],
        out_specs=[pl.BlockSpec((NB, H), nb), pl.BlockSpec((NB, 16), nb),
                   pl.BlockSpec((NB, CW), nb), pl.BlockSpec((NB, CW), nb)],
        out_shape=[jax.ShapeDtypeStruct((NN, H), f32),
                   jax.ShapeDtypeStruct((NN, 16), f32),
                   jax.ShapeDtypeStruct((NN, CW), f32),
                   jax.ShapeDtypeStruct((NN, CW), f32)],
    )(hh, cp, acc, acc, accB, accB, st, st, stB, stB,
      w["Wn1a"], w["Wn1b"], w["bn1"],
      w["Wn2"], w["bn2"], wn["W1a"], wn["W1b"])


def _node_final_body(hh, a0, a1, a2, a3, Wn1a, Wn1b, bn1, Wn2, bn2,
                     Weo, beo, ps_o):
    agg = (a0[0] + a1[0]) + (a2[0] + a3[0])
    t = hh[...] @ Wn1a[...] + agg @ Wn1b[...] + bn1[...]
    t = t * jax.nn.sigmoid(t)
    hn = hh[...] + t @ Wn2[...] + bn2[...]
    eo = hn @ Weo[...] + beo[...]

    @pl.when(pl.program_id(0) == 0)
    def _():
        ps_o[...] = jnp.zeros((1, H), f32)

    ps_o[...] += jnp.sum(eo, axis=0, keepdims=True)


def _node_final(hh, acc, accB, w, Weo, beo):
    nb = lambda i: (i, 0)
    a0 = lambda i: (0, i, 0)
    a1 = lambda i: (1, i, 0)
    w0 = lambda i: (0, 0)
    return pl.pallas_call(
        _node_final_body,
        grid=(NN // NB,),
        in_specs=[
            pl.BlockSpec((NB, H), nb),
            pl.BlockSpec((1, NB, H), a0),
            pl.BlockSpec((1, NB, H), a1),
            pl.BlockSpec((1, NB, H), a0),
            pl.BlockSpec((1, NB, H), a1),
            pl.BlockSpec((H, H), w0),
            pl.BlockSpec((H, H), w0),
            pl.BlockSpec((1, H), w0),
            pl.BlockSpec((H, H), w0),
            pl.BlockSpec((1, H), w0),
            pl.BlockSpec((H, H), w0),
            pl.BlockSpec((1, H), w0),
        ],
        out_specs=pl.BlockSpec((1, H), w0),
        out_shape=jax.ShapeDtypeStruct((1, H), f32),
    )(hh, acc, acc, accB, accB, w["Wn1a"], w["Wn1b"], w["bn1"],
      w["Wn2"], w["bn2"], Weo, beo)


def _emb_body(h_in, xp, We, be_, W1a, W1b, hh_o, t1_o, t2_o):
    hh = h_in[...] @ We[...] + be_[...]
    hh_o[...] = hh
    t1_o[...] = jnp.concatenate([hh @ W1a[...], xp[...]], axis=1)
    t2_o[...] = jnp.concatenate([hh @ W1b[...], xp[...]], axis=1)


def _emb(h, xp, We, be_, W1a, W1b):
    nb = lambda i: (i, 0)
    w0 = lambda i: (0, 0)
    return pl.pallas_call(
        _emb_body,
        grid=(NN // NB,),
        in_specs=[
            pl.BlockSpec((NB, H), nb),
            pl.BlockSpec((NB, 16), nb),
            pl.BlockSpec((H, H), w0),
            pl.BlockSpec((1, H), w0),
            pl.BlockSpec((H, H), w0),
            pl.BlockSpec((H, H), w0),
        ],
        out_specs=[pl.BlockSpec((NB, H), nb), pl.BlockSpec((NB, CW), nb),
                   pl.BlockSpec((NB, CW), nb)],
        out_shape=[jax.ShapeDtypeStruct((NN, H), f32),
                   jax.ShapeDtypeStruct((NN, CW), f32),
                   jax.ShapeDtypeStruct((NN, CW), f32)],
    )(h, xp, We, be_, W1a, W1b)


def _head_body(ps, Wm1, bm1, Wm2, bm2, o_ref):
    hm = ps[...] * (1.0 / NN)
    z = hm @ Wm1[...] + bm1[...]
    z = jnp.maximum(z, 0.0)
    o_ref[...] = z @ Wm2[...] + bm2[...]


def _head(ps, Wm1, bm1, Wm2, bm2):
    w0 = lambda i: (0, 0)
    return pl.pallas_call(
        _head_body,
        grid=(1,),
        in_specs=[
            pl.BlockSpec((1, H), w0),
            pl.BlockSpec((H, H), w0),
            pl.BlockSpec((1, H), w0),
            pl.BlockSpec((H, H), w0),
            pl.BlockSpec((1, H), w0),
        ],
        out_specs=pl.BlockSpec((1, H), w0),
        out_shape=jax.ShapeDtypeStruct((1, H), f32),
    )(ps, Wm1, bm1, Wm2, bm2)


def _layer_views(p):
    W1 = p["edge1"]["W"]
    return dict(
        W1a=W1[:H],
        W1b=W1[H:2 * H],
        ew=(W1[2 * H:2 * H + 1],
            p["edge1"]["b"].reshape(1, H),
            p["edge2"]["W"].astype(bf16),
            p["edge2"]["b"].reshape(1, H),
            p["att"]["W"].T,
            jnp.broadcast_to(p["att"]["b"].reshape(1, 1), (1, H)),
            p["coord1"]["W"].astype(bf16),
            p["coord1"]["b"].reshape(1, H),
            p["coord2"]["W"].T),
        Wn1a=p["node1"]["W"][:H],
        Wn1b=p["node1"]["W"][H:],
        bn1=p["node1"]["b"].reshape(1, H),
        Wn2=p["node2"]["W"],
        bn2=p["node2"]["b"].reshape(1, H),
    )


def kernel(h, x, edges, params):
    rowrA = edges[0][:EA].reshape(-1, CH)
    colrA = edges[1][:EA].reshape(-1, CH)
    rowrB = edges[0][EA:].reshape(-1, CH)
    colrB = edges[1][EA:].reshape(-1, CH)
    xp = jnp.pad(x, ((0, 0), (0, 13)))
    z128 = jnp.zeros((NPAD, H), f32)
    z16 = jnp.zeros((NPAD, 16), f32)
    lws = [_layer_views(p) for p in params["layers"]]
    hh, t1, t2 = _emb(h, xp, params["emb_in"]["W"],
                      params["emb_in"]["b"].reshape(1, H),
                      lws[0]["W1a"], lws[0]["W1b"])
    cp = xp
    ps = None
    for l in range(4):
        m1B, cdB = _sc_gather(t1, t2, rowrB, colrB, EB)
        m1A, cdA = _sc_gather(t1, t2, rowrA, colrA, EA)
        moB, trB = _edge_tc(m1B, cdB, lws[l]["ew"], EB)
        accB, stB = _sc_scatter(moB, trB, rowrB, z128, z16, EB)
        moA, trA = _edge_tc(m1A, cdA, lws[l]["ew"], EA)
        accA, stA = _sc_scatter(moA, trA, rowrA, z128, z16, EA)
        if l < 3:
            hh, cp, t1, t2 = _node_tc(hh, cp, accA, stA, accB, stB,
                                      lws[l], lws[l + 1])
        else:
            ps = _node_final(hh, accA, accB, lws[l],
                             params["emb_out"]["W"],
                             params["emb_out"]["b"].reshape(1, H))
    o = _head(ps, params["mlp1"]["W"], params["mlp1"]["b"].reshape(1, H),
              jnp.pad(params["mlp2"]["W"], ((0, 0), (0, H - 20))),
              jnp.pad(params["mlp2"]["b"], (0, H - 20)).reshape(1, H))
    return o[0, :20]
